# baseline (speedup 1.0000x reference)
import jax
import jax.numpy as jnp
from jax.experimental import pallas as pl

B, N, M = 4, 8192, 2048
QK, KNN, GEO = 64, 16, 18
T = 512
NT = N // T
EPS = 1e-5
NEG_BIG = -3.0e38


def _p1_kernel(fh_ref, fl_ref, w_ref, b_ref, yh_ref, yl_ref,
               sh_ref, qh_ref, sl_ref, ql_ref):
    w = w_ref[...]
    bcol = b_ref[...]
    @pl.when(pl.program_id(0) == 0)
    def _():
        for r in (sh_ref, qh_ref, sl_ref, ql_ref):
            r[...] = jnp.zeros_like(r)
    yh = jax.lax.dot_general(w, fh_ref[0], (((1,), (0,)), ((), ())),
                             preferred_element_type=jnp.float32) + bcol
    yh_ref[0] = yh
    sh_ref[...] += jnp.sum(yh, axis=1, keepdims=True)
    qh_ref[...] += jnp.sum(yh * yh, axis=1, keepdims=True)
    yl = jax.lax.dot_general(w, fl_ref[0], (((1,), (0,)), ((), ())),
                             preferred_element_type=jnp.float32) + bcol
    yl_ref[0] = yl
    sl_ref[...] += jnp.sum(yl, axis=1, keepdims=True)
    ql_ref[...] += jnp.sum(yl * yl, axis=1, keepdims=True)


def _p2_kernel(yh_ref, yl_ref, w_ref, b_ref, f_ref, y2h_ref, y2l_ref,
               sh_ref, qh_ref, sl_ref, ql_ref):
    w = w_ref[...]
    bcol = b_ref[...]
    s_h = f_ref[:, 0:1]
    t_h = f_ref[:, 1:2]
    s_l = f_ref[:, 2:3]
    t_l = f_ref[:, 3:4]
    @pl.when(pl.program_id(0) == 0)
    def _():
        for r in (sh_ref, qh_ref, sl_ref, ql_ref):
            r[...] = jnp.zeros_like(r)
    hh = jnp.maximum(yh_ref[0] * s_h + t_h, 0.0)
    y2 = jax.lax.dot_general(w, hh, (((1,), (0,)), ((), ())),
                             preferred_element_type=jnp.float32) + bcol
    y2h_ref[0] = y2
    sh_ref[...] += jnp.sum(y2, axis=1, keepdims=True)
    qh_ref[...] += jnp.sum(y2 * y2, axis=1, keepdims=True)
    hl = jnp.maximum(yl_ref[0] * s_l + t_l, 0.0)
    y2l = jax.lax.dot_general(w, hl, (((1,), (0,)), ((), ())),
                              preferred_element_type=jnp.float32) + bcol
    y2l_ref[0] = y2l
    sl_ref[...] += jnp.sum(y2l, axis=1, keepdims=True)
    ql_ref[...] += jnp.sum(y2l * y2l, axis=1, keepdims=True)


def _p3_kernel(y2h_ref, y2l_ref, val_ref, f_ref, wq_ref, wk_ref,
               wsc_ref, wsh_ref, wbd_ref, bb_ref,
               q_ref, k_ref, ybd_ref, sb_ref, qb_ref):
    s_h = f_ref[:, 0:1]
    t_h = f_ref[:, 1:2]
    s_l = f_ref[:, 2:3]
    t_l = f_ref[:, 3:4]
    bq = f_ref[:, 4:5]
    bk = f_ref[:, 5:6]
    bsc = f_ref[:, 6:7]
    bsh = f_ref[:, 7:8]
    wq = wq_ref[...]
    wk = wk_ref[...]
    wsc = wsc_ref[...]
    wsh = wsh_ref[...]
    wbd = wbd_ref[...]
    bbd = bb_ref[...]
    @pl.when(pl.program_id(0) == 0)
    def _():
        sb_ref[...] = jnp.zeros_like(sb_ref)
        qb_ref[...] = jnp.zeros_like(qb_ref)
    gh = jnp.maximum(y2h_ref[0] * s_h + t_h, 0.0)
    q_ref[0] = jax.lax.dot_general(wq, gh, (((1,), (0,)), ((), ())),
                                   preferred_element_type=jnp.float32) + bq
    ybd = jax.lax.dot_general(wbd, gh, (((1,), (0,)), ((), ())),
                              preferred_element_type=jnp.float32) + bbd
    ybd_ref[0] = ybd
    sb_ref[...] += jnp.sum(ybd, axis=1, keepdims=True)
    qb_ref[...] += jnp.sum(ybd * ybd, axis=1, keepdims=True)
    gl = jnp.maximum(y2l_ref[0] * s_l + t_l, 0.0)
    v = val_ref[0]
    sc = jax.lax.dot_general(wsc, v, (((1,), (0,)), ((), ())),
                             preferred_element_type=jnp.float32) + bsc
    sf = jax.lax.dot_general(wsh, v, (((1,), (0,)), ((), ())),
                             preferred_element_type=jnp.float32) + bsh
    gl = gl * (sc + 1.0) + sf
    k_ref[0] = jax.lax.dot_general(wk, gl, (((1,), (0,)), ((), ())),
                                   preferred_element_type=jnp.float32) + bk


def _chunk_gather_rows(idx_t, w, src):
    acc = jnp.zeros(idx_t.shape, jnp.float32)
    for cc in range(M // 128):
        g = jnp.take_along_axis(src[:, cc * 128:(cc + 1) * 128], w, axis=1)
        acc = jnp.where((idx_t >> 7) == cc, g, acc)
    return acc


def _chunk_gather_bcast(ch, wj, row):
    acc = jnp.zeros(ch.shape, jnp.float32)
    for cc in range(M // 128):
        src = jnp.broadcast_to(row[:, cc * 128:(cc + 1) * 128], (KNN, 128))
        g = jnp.take_along_axis(src, wj, axis=1)
        acc = jnp.where(ch == cc, g, acc)
    return acc


def _pk_kernel(xh_ref, xl_ref, q_ref, k_ref, ybd_ref, f_ref, wbd2_ref,
               w1r_ref, kidx_ref, sg_ref, bdy_ref, rel_ref, sr_ref, qr_ref):
    xh = xh_ref[0]
    xl = xl_ref[0]
    g = jax.lax.dot_general(xh, xl, (((0,), (0,)), ((), ())),
                            preferred_element_type=jnp.float32)
    bn2 = jnp.sum(xl * xl, axis=0, keepdims=True)
    neg = 2.0 * g - bn2
    iota = jax.lax.broadcasted_iota(jnp.int32, (T, M), 1)
    cols = []
    for _ in range(KNN):
        idxm = jnp.argmax(neg, axis=1, keepdims=True)
        neg = jnp.where(iota == idxm, NEG_BIG, neg)
        cols.append(idxm)
    idx_t = jnp.concatenate(cols, axis=1)
    s = jax.lax.dot_general(q_ref[0], k_ref[0], (((0,), (0,)), ((), ())),
                            preferred_element_type=jnp.float32)
    w = idx_t & 127
    sg = _chunk_gather_rows(idx_t, w, s)
    idx_jt = idx_t.T
    sg_ref[0] = sg.T
    kidx_ref[0] = idx_jt
    ch = idx_jt >> 7
    wj = idx_jt & 127
    rels = []
    for c in range(3):
        gxyz = _chunk_gather_bcast(ch, wj, xl[c:c + 1])
        rels.append((jnp.broadcast_to(xh[c:c + 1], (KNN, T)) - gxyz)[None])
    rel3 = jnp.concatenate(rels, axis=0)
    rel_ref[0] = rel3.reshape(3 * KNN, T)
    y1r = jax.lax.dot_general(w1r_ref[...], rel3.reshape(3, KNN * T),
                              (((1,), (0,)), ((), ())),
                              preferred_element_type=jnp.float32)
    y1r = y1r + f_ref[:, 2:3]
    @pl.when((pl.program_id(0) == 0) & (pl.program_id(1) == 0))
    def _():
        sr_ref[...] = jnp.zeros_like(sr_ref)
        qr_ref[...] = jnp.zeros_like(qr_ref)
    sr_ref[...] += jnp.sum(y1r, axis=1, keepdims=True)
    qr_ref[...] += jnp.sum(y1r * y1r, axis=1, keepdims=True)
    s_b = f_ref[:32, 0:1]
    t_b = f_ref[:32, 1:2]
    hbd = jnp.maximum(ybd_ref[0] * s_b + t_b, 0.0)
    logit = jax.lax.dot_general(wbd2_ref[...], hbd, (((1,), (0,)), ((), ())),
                                preferred_element_type=jnp.float32)
    bdy_ref[0] = jax.nn.sigmoid(logit + f_ref[0, 3])


def _p6_kernel(rel_ref, sg_ref, kidx_ref, q_ref, val_ref, f_ref,
               w1r_ref, w2q_ref, rec_ref):
    rel = rel_ref[0].reshape(3, KNN, T).reshape(3, KNN * T)
    y1r = jax.lax.dot_general(w1r_ref[...], rel, (((1,), (0,)), ((), ())),
                              preferred_element_type=jnp.float32) + f_ref[:, 2:3]
    h2 = jnp.maximum(y1r * f_ref[:, 0:1] + f_ref[:, 1:2], 0.0)
    h23 = h2.reshape(QK, KNN, T)
    q2 = jax.lax.dot_general(w2q_ref[...], q_ref[0], (((1,), (0,)), ((), ())),
                             preferred_element_type=jnp.float32)
    pos = jnp.sum(h23 * q2[:, None, :], axis=0)
    logits = (sg_ref[0] + pos) * 0.125
    mx = jnp.max(logits, axis=0, keepdims=True)
    e = jnp.exp(logits - mx)
    attn = e / jnp.sum(e, axis=0, keepdims=True)
    idx_jt = kidx_ref[0]
    ch = idx_jt >> 7
    wj = idx_jt & 127
    recs = []
    for c in range(6):
        vg = _chunk_gather_bcast(ch, wj, val_ref[0, c:c + 1])
        recs.append(jnp.sum(attn * vg, axis=0, keepdims=True))
    rec_ref[0] = jnp.concatenate(recs, axis=0)


def _fold(ssum, ssq, n, gamma, beta):
    mu = ssum[:, 0] / n
    var = ssq[:, 0] / n - mu * mu
    s = gamma / jnp.sqrt(var + EPS)
    return s, beta - mu * s


def _col(v):
    return v.reshape(-1, 1)


def kernel(xyz_hr, xyz_lr, val_lr, geo_blobs_hr, geo_blobs_lr,
           rgb_blobs_hr, rgb_blobs_lr, params):
    p = params
    f32 = jnp.float32
    feat_hr = jnp.concatenate([geo_blobs_hr, rgb_blobs_hr], axis=1)
    feat_lr = jnp.concatenate([geo_blobs_lr, rgb_blobs_lr], axis=1)

    sds = jax.ShapeDtypeStruct
    stat = sds((QK, 1), f32)

    def _c(shape):
        return pl.BlockSpec(shape, lambda b: (0,) * len(shape))

    def _b3(c, n):
        return pl.BlockSpec((1, c, n), lambda b: (b, 0, 0))

    st_spec = pl.BlockSpec((QK, 1), lambda b: (0, 0))

    y1h, y1l, sh, qh, sl, ql = pl.pallas_call(
        _p1_kernel,
        grid=(B,),
        in_specs=[_b3(GEO, N), _b3(GEO, M), _c((QK, GEO)), _c((QK, 1))],
        out_specs=[_b3(QK, N), _b3(QK, M), st_spec, st_spec, st_spec,
                   st_spec],
        out_shape=(sds((B, QK, N), f32), sds((B, QK, M), f32),
                   stat, stat, stat, stat),
    )(feat_hr, feat_lr, p['ge_w1'], _col(p['ge_b1']))
    s1h, t1h = _fold(sh, qh, B * N, p['ge_g1'], p['ge_be1'])
    s1l, t1l = _fold(sl, ql, B * M, p['ge_g1'], p['ge_be1'])

    fold2 = jnp.stack([s1h, t1h, s1l, t1l], axis=1)
    y2h, y2l, sh, qh, sl, ql = pl.pallas_call(
        _p2_kernel,
        grid=(B,),
        in_specs=[_b3(QK, N), _b3(QK, M), _c((QK, QK)), _c((QK, 1)),
                  _c((QK, 4))],
        out_specs=[_b3(QK, N), _b3(QK, M), st_spec, st_spec, st_spec,
                   st_spec],
        out_shape=(sds((B, QK, N), f32), sds((B, QK, M), f32),
                   stat, stat, stat, stat),
    )(y1h, y1l, p['ge_w2'], _col(p['ge_b2']), fold2)
    s2h, t2h = _fold(sh, qh, B * N, p['ge_g2'], p['ge_be2'])
    s2l, t2l = _fold(sl, ql, B * M, p['ge_g2'], p['ge_be2'])

    fold3 = jnp.stack([s2h, t2h, s2l, t2l, p['q_b'], p['k_b'],
                       p['sc_b'], p['sh_b']], axis=1)
    q, k, ybd, sb, qb = pl.pallas_call(
        _p3_kernel,
        grid=(B,),
        in_specs=[_b3(QK, N), _b3(QK, M), _b3(6, M), _c((QK, 8)),
                  _c((QK, QK)), _c((QK, QK)), _c((QK, 6)), _c((QK, 6)),
                  _c((32, QK)), _c((32, 1))],
        out_specs=[_b3(QK, N), _b3(QK, M), _b3(32, N),
                   pl.BlockSpec((32, 1), lambda b: (0, 0)),
                   pl.BlockSpec((32, 1), lambda b: (0, 0))],
        out_shape=(sds((B, QK, N), f32), sds((B, QK, M), f32),
                   sds((B, 32, N), f32), sds((32, 1), f32), sds((32, 1), f32)),
    )(y2h, y2l, val_lr, fold3, p['q_w'], p['k_w'], p['sc_w'], p['sh_w'],
      p['bd_w1'], _col(p['bd_b1']))
    sbd, tbd = _fold(sb, qb, B * N, p['bd_g1'], p['bd_be1'])

    pad = jnp.zeros((QK - 32,), f32)
    foldk = jnp.stack([jnp.concatenate([sbd, pad]),
                       jnp.concatenate([tbd, pad]),
                       p['rp_b1'],
                       jnp.full((QK,), p['bd_b2'][0], f32)], axis=1)
    kidx, sg, bdy, rel, sr, qr = pl.pallas_call(
        _pk_kernel,
        grid=(B, NT),
        in_specs=[
            pl.BlockSpec((1, 3, T), lambda b, i: (b, 0, i)),
            pl.BlockSpec((1, 3, M), lambda b, i: (b, 0, 0)),
            pl.BlockSpec((1, QK, T), lambda b, i: (b, 0, i)),
            pl.BlockSpec((1, QK, M), lambda b, i: (b, 0, 0)),
            pl.BlockSpec((1, 32, T), lambda b, i: (b, 0, i)),
            pl.BlockSpec((QK, 4), lambda b, i: (0, 0)),
            pl.BlockSpec((1, 32), lambda b, i: (0, 0)),
            pl.BlockSpec((QK, 3), lambda b, i: (0, 0)),
        ],
        out_specs=[
            pl.BlockSpec((1, KNN, T), lambda b, i: (b, 0, i)),
            pl.BlockSpec((1, KNN, T), lambda b, i: (b, 0, i)),
            pl.BlockSpec((1, 1, T), lambda b, i: (b, 0, i)),
            pl.BlockSpec((1, 3 * KNN, T), lambda b, i: (b, 0, i)),
            pl.BlockSpec((QK, 1), lambda b, i: (0, 0)),
            pl.BlockSpec((QK, 1), lambda b, i: (0, 0)),
        ],
        out_shape=(sds((B, KNN, N), jnp.int32), sds((B, KNN, N), f32),
                   sds((B, 1, N), f32), sds((B, 3 * KNN, N), f32),
                   stat, stat),
    )(xyz_hr, xyz_lr, q, k, ybd, foldk, p['bd_w2'], p['rp_w1'])
    sr1, tr1 = _fold(sr, qr, B * N * KNN, p['rp_g1'], p['rp_be1'])

    fold5 = jnp.stack([sr1, tr1, p['rp_b1'], p['rp_b2']], axis=1)
    w2q = p['rp_w2'].T
    q2w = pl.pallas_call(
        _p6_kernel,
        grid=(B, NT),
        in_specs=[
            pl.BlockSpec((1, 3 * KNN, T), lambda b, i: (b, 0, i)),
            pl.BlockSpec((1, KNN, T), lambda b, i: (b, 0, i)),
            pl.BlockSpec((1, KNN, T), lambda b, i: (b, 0, i)),
            pl.BlockSpec((1, QK, T), lambda b, i: (b, 0, i)),
            pl.BlockSpec((1, 6, M), lambda b, i: (b, 0, 0)),
            pl.BlockSpec((QK, 4), lambda b, i: (0, 0)),
            pl.BlockSpec((QK, 3), lambda b, i: (0, 0)),
            pl.BlockSpec((QK, QK), lambda b, i: (0, 0)),
        ],
        out_specs=pl.BlockSpec((1, 6, T), lambda b, i: (b, 0, i)),
        out_shape=sds((B, 6, N), f32),
    )(rel, sg, kidx, q, val_lr, fold5, p['rp_w1'], w2q)

    return (q2w, bdy)


# speedup vs baseline: 1.0140x; 1.0140x over previous
import jax
import jax.numpy as jnp
from jax.experimental import pallas as pl

B, N, M = 4, 8192, 2048
QK, KNN, GEO = 64, 16, 18
T = 256
NT = N // T
EPS = 1e-5
NEG_BIG = -3.0e38


def _p1_kernel(fh_ref, fl_ref, w_ref, b_ref, yh_ref, yl_ref,
               sh_ref, qh_ref, sl_ref, ql_ref):
    w = w_ref[...]
    bcol = b_ref[...]
    @pl.when(pl.program_id(0) == 0)
    def _():
        for r in (sh_ref, qh_ref, sl_ref, ql_ref):
            r[...] = jnp.zeros_like(r)
    yh = jax.lax.dot_general(w, fh_ref[0], (((1,), (0,)), ((), ())),
                             preferred_element_type=jnp.float32) + bcol
    yh_ref[0] = yh
    sh_ref[...] += jnp.sum(yh, axis=1, keepdims=True)
    qh_ref[...] += jnp.sum(yh * yh, axis=1, keepdims=True)
    yl = jax.lax.dot_general(w, fl_ref[0], (((1,), (0,)), ((), ())),
                             preferred_element_type=jnp.float32) + bcol
    yl_ref[0] = yl
    sl_ref[...] += jnp.sum(yl, axis=1, keepdims=True)
    ql_ref[...] += jnp.sum(yl * yl, axis=1, keepdims=True)


def _p2_kernel(yh_ref, yl_ref, w_ref, b_ref, f_ref, y2h_ref, y2l_ref,
               sh_ref, qh_ref, sl_ref, ql_ref):
    w = w_ref[...]
    bcol = b_ref[...]
    s_h = f_ref[:, 0:1]
    t_h = f_ref[:, 1:2]
    s_l = f_ref[:, 2:3]
    t_l = f_ref[:, 3:4]
    @pl.when(pl.program_id(0) == 0)
    def _():
        for r in (sh_ref, qh_ref, sl_ref, ql_ref):
            r[...] = jnp.zeros_like(r)
    hh = jnp.maximum(yh_ref[0] * s_h + t_h, 0.0)
    y2 = jax.lax.dot_general(w, hh, (((1,), (0,)), ((), ())),
                             preferred_element_type=jnp.float32) + bcol
    y2h_ref[0] = y2
    sh_ref[...] += jnp.sum(y2, axis=1, keepdims=True)
    qh_ref[...] += jnp.sum(y2 * y2, axis=1, keepdims=True)
    hl = jnp.maximum(yl_ref[0] * s_l + t_l, 0.0)
    y2l = jax.lax.dot_general(w, hl, (((1,), (0,)), ((), ())),
                              preferred_element_type=jnp.float32) + bcol
    y2l_ref[0] = y2l
    sl_ref[...] += jnp.sum(y2l, axis=1, keepdims=True)
    ql_ref[...] += jnp.sum(y2l * y2l, axis=1, keepdims=True)


def _p3_kernel(y2h_ref, y2l_ref, val_ref, f_ref, wq_ref, wk_ref,
               wsc_ref, wsh_ref, wbd_ref, bb_ref,
               q_ref, k_ref, ybd_ref, sb_ref, qb_ref):
    s_h = f_ref[:, 0:1]
    t_h = f_ref[:, 1:2]
    s_l = f_ref[:, 2:3]
    t_l = f_ref[:, 3:4]
    bq = f_ref[:, 4:5]
    bk = f_ref[:, 5:6]
    bsc = f_ref[:, 6:7]
    bsh = f_ref[:, 7:8]
    wq = wq_ref[...]
    wk = wk_ref[...]
    wsc = wsc_ref[...]
    wsh = wsh_ref[...]
    wbd = wbd_ref[...]
    bbd = bb_ref[...]
    @pl.when(pl.program_id(0) == 0)
    def _():
        sb_ref[...] = jnp.zeros_like(sb_ref)
        qb_ref[...] = jnp.zeros_like(qb_ref)
    gh = jnp.maximum(y2h_ref[0] * s_h + t_h, 0.0)
    q_ref[0] = jax.lax.dot_general(wq, gh, (((1,), (0,)), ((), ())),
                                   preferred_element_type=jnp.float32) + bq
    ybd = jax.lax.dot_general(wbd, gh, (((1,), (0,)), ((), ())),
                              preferred_element_type=jnp.float32) + bbd
    ybd_ref[0] = ybd
    sb_ref[...] += jnp.sum(ybd, axis=1, keepdims=True)
    qb_ref[...] += jnp.sum(ybd * ybd, axis=1, keepdims=True)
    gl = jnp.maximum(y2l_ref[0] * s_l + t_l, 0.0)
    v = val_ref[0]
    sc = jax.lax.dot_general(wsc, v, (((1,), (0,)), ((), ())),
                             preferred_element_type=jnp.float32) + bsc
    sf = jax.lax.dot_general(wsh, v, (((1,), (0,)), ((), ())),
                             preferred_element_type=jnp.float32) + bsh
    gl = gl * (sc + 1.0) + sf
    k_ref[0] = jax.lax.dot_general(wk, gl, (((1,), (0,)), ((), ())),
                                   preferred_element_type=jnp.float32) + bk


def _chunk_gather_rows(idx_t, w, src):
    acc = jnp.zeros(idx_t.shape, jnp.float32)
    for cc in range(M // 128):
        g = jnp.take_along_axis(src[:, cc * 128:(cc + 1) * 128], w, axis=1)
        acc = jnp.where((idx_t >> 7) == cc, g, acc)
    return acc


def _chunk_gather_bcast(ch, wj, row):
    acc = jnp.zeros(ch.shape, jnp.float32)
    for cc in range(M // 128):
        src = jnp.broadcast_to(row[:, cc * 128:(cc + 1) * 128], (KNN, 128))
        g = jnp.take_along_axis(src, wj, axis=1)
        acc = jnp.where(ch == cc, g, acc)
    return acc


def _pk_kernel(xh_ref, xl_ref, q_ref, k_ref, ybd_ref, f_ref, wbd2_ref,
               w1r_ref, kidx_ref, sg_ref, bdy_ref, rel_ref, sr_ref, qr_ref):
    xh = xh_ref[0]
    xl = xl_ref[0]
    g = jax.lax.dot_general(xh, xl, (((0,), (0,)), ((), ())),
                            preferred_element_type=jnp.float32)
    bn2 = jnp.sum(xl * xl, axis=0, keepdims=True)
    neg = 2.0 * g - bn2
    iota = jax.lax.broadcasted_iota(jnp.int32, (T, M), 1)
    cols = []
    for _ in range(KNN):
        idxm = jnp.argmax(neg, axis=1, keepdims=True)
        neg = jnp.where(iota == idxm, NEG_BIG, neg)
        cols.append(idxm)
    idx_t = jnp.concatenate(cols, axis=1)
    s = jax.lax.dot_general(q_ref[0], k_ref[0], (((0,), (0,)), ((), ())),
                            preferred_element_type=jnp.float32)
    w = idx_t & 127
    sg = _chunk_gather_rows(idx_t, w, s)
    idx_jt = idx_t.T
    sg_ref[0] = sg.T
    kidx_ref[0] = idx_jt
    ch = idx_jt >> 7
    wj = idx_jt & 127
    rels = []
    for c in range(3):
        gxyz = _chunk_gather_bcast(ch, wj, xl[c:c + 1])
        rels.append((jnp.broadcast_to(xh[c:c + 1], (KNN, T)) - gxyz)[None])
    rel3 = jnp.concatenate(rels, axis=0)
    rel_ref[0] = rel3.reshape(3 * KNN, T)
    y1r = jax.lax.dot_general(w1r_ref[...], rel3.reshape(3, KNN * T),
                              (((1,), (0,)), ((), ())),
                              preferred_element_type=jnp.float32)
    y1r = y1r + f_ref[:, 2:3]
    @pl.when((pl.program_id(0) == 0) & (pl.program_id(1) == 0))
    def _():
        sr_ref[...] = jnp.zeros_like(sr_ref)
        qr_ref[...] = jnp.zeros_like(qr_ref)
    sr_ref[...] += jnp.sum(y1r, axis=1, keepdims=True)
    qr_ref[...] += jnp.sum(y1r * y1r, axis=1, keepdims=True)
    s_b = f_ref[:32, 0:1]
    t_b = f_ref[:32, 1:2]
    hbd = jnp.maximum(ybd_ref[0] * s_b + t_b, 0.0)
    logit = jax.lax.dot_general(wbd2_ref[...], hbd, (((1,), (0,)), ((), ())),
                                preferred_element_type=jnp.float32)
    bdy_ref[0] = jax.nn.sigmoid(logit + f_ref[0, 3])


def _p6_kernel(rel_ref, sg_ref, kidx_ref, q_ref, val_ref, f_ref,
               w1r_ref, w2q_ref, rec_ref):
    rel = rel_ref[0].reshape(3, KNN, T).reshape(3, KNN * T)
    y1r = jax.lax.dot_general(w1r_ref[...], rel, (((1,), (0,)), ((), ())),
                              preferred_element_type=jnp.float32) + f_ref[:, 2:3]
    h2 = jnp.maximum(y1r * f_ref[:, 0:1] + f_ref[:, 1:2], 0.0)
    h23 = h2.reshape(QK, KNN, T)
    q2 = jax.lax.dot_general(w2q_ref[...], q_ref[0], (((1,), (0,)), ((), ())),
                             preferred_element_type=jnp.float32)
    pos = jnp.sum(h23 * q2[:, None, :], axis=0)
    logits = (sg_ref[0] + pos) * 0.125
    mx = jnp.max(logits, axis=0, keepdims=True)
    e = jnp.exp(logits - mx)
    attn = e / jnp.sum(e, axis=0, keepdims=True)
    idx_jt = kidx_ref[0]
    ch = idx_jt >> 7
    wj = idx_jt & 127
    recs = []
    for c in range(6):
        vg = _chunk_gather_bcast(ch, wj, val_ref[0, c:c + 1])
        recs.append(jnp.sum(attn * vg, axis=0, keepdims=True))
    rec_ref[0] = jnp.concatenate(recs, axis=0)


def _fold(ssum, ssq, n, gamma, beta):
    mu = ssum[:, 0] / n
    var = ssq[:, 0] / n - mu * mu
    s = gamma / jnp.sqrt(var + EPS)
    return s, beta - mu * s


def _col(v):
    return v.reshape(-1, 1)


def kernel(xyz_hr, xyz_lr, val_lr, geo_blobs_hr, geo_blobs_lr,
           rgb_blobs_hr, rgb_blobs_lr, params):
    p = params
    f32 = jnp.float32
    feat_hr = jnp.concatenate([geo_blobs_hr, rgb_blobs_hr], axis=1)
    feat_lr = jnp.concatenate([geo_blobs_lr, rgb_blobs_lr], axis=1)

    sds = jax.ShapeDtypeStruct
    stat = sds((QK, 1), f32)

    def _c(shape):
        return pl.BlockSpec(shape, lambda b: (0,) * len(shape))

    def _b3(c, n):
        return pl.BlockSpec((1, c, n), lambda b: (b, 0, 0))

    st_spec = pl.BlockSpec((QK, 1), lambda b: (0, 0))

    y1h, y1l, sh, qh, sl, ql = pl.pallas_call(
        _p1_kernel,
        grid=(B,),
        in_specs=[_b3(GEO, N), _b3(GEO, M), _c((QK, GEO)), _c((QK, 1))],
        out_specs=[_b3(QK, N), _b3(QK, M), st_spec, st_spec, st_spec,
                   st_spec],
        out_shape=(sds((B, QK, N), f32), sds((B, QK, M), f32),
                   stat, stat, stat, stat),
    )(feat_hr, feat_lr, p['ge_w1'], _col(p['ge_b1']))
    s1h, t1h = _fold(sh, qh, B * N, p['ge_g1'], p['ge_be1'])
    s1l, t1l = _fold(sl, ql, B * M, p['ge_g1'], p['ge_be1'])

    fold2 = jnp.stack([s1h, t1h, s1l, t1l], axis=1)
    y2h, y2l, sh, qh, sl, ql = pl.pallas_call(
        _p2_kernel,
        grid=(B,),
        in_specs=[_b3(QK, N), _b3(QK, M), _c((QK, QK)), _c((QK, 1)),
                  _c((QK, 4))],
        out_specs=[_b3(QK, N), _b3(QK, M), st_spec, st_spec, st_spec,
                   st_spec],
        out_shape=(sds((B, QK, N), f32), sds((B, QK, M), f32),
                   stat, stat, stat, stat),
    )(y1h, y1l, p['ge_w2'], _col(p['ge_b2']), fold2)
    s2h, t2h = _fold(sh, qh, B * N, p['ge_g2'], p['ge_be2'])
    s2l, t2l = _fold(sl, ql, B * M, p['ge_g2'], p['ge_be2'])

    fold3 = jnp.stack([s2h, t2h, s2l, t2l, p['q_b'], p['k_b'],
                       p['sc_b'], p['sh_b']], axis=1)
    q, k, ybd, sb, qb = pl.pallas_call(
        _p3_kernel,
        grid=(B,),
        in_specs=[_b3(QK, N), _b3(QK, M), _b3(6, M), _c((QK, 8)),
                  _c((QK, QK)), _c((QK, QK)), _c((QK, 6)), _c((QK, 6)),
                  _c((32, QK)), _c((32, 1))],
        out_specs=[_b3(QK, N), _b3(QK, M), _b3(32, N),
                   pl.BlockSpec((32, 1), lambda b: (0, 0)),
                   pl.BlockSpec((32, 1), lambda b: (0, 0))],
        out_shape=(sds((B, QK, N), f32), sds((B, QK, M), f32),
                   sds((B, 32, N), f32), sds((32, 1), f32), sds((32, 1), f32)),
    )(y2h, y2l, val_lr, fold3, p['q_w'], p['k_w'], p['sc_w'], p['sh_w'],
      p['bd_w1'], _col(p['bd_b1']))
    sbd, tbd = _fold(sb, qb, B * N, p['bd_g1'], p['bd_be1'])

    pad = jnp.zeros((QK - 32,), f32)
    foldk = jnp.stack([jnp.concatenate([sbd, pad]),
                       jnp.concatenate([tbd, pad]),
                       p['rp_b1'],
                       jnp.full((QK,), p['bd_b2'][0], f32)], axis=1)
    kidx, sg, bdy, rel, sr, qr = pl.pallas_call(
        _pk_kernel,
        grid=(B, NT),
        in_specs=[
            pl.BlockSpec((1, 3, T), lambda b, i: (b, 0, i)),
            pl.BlockSpec((1, 3, M), lambda b, i: (b, 0, 0)),
            pl.BlockSpec((1, QK, T), lambda b, i: (b, 0, i)),
            pl.BlockSpec((1, QK, M), lambda b, i: (b, 0, 0)),
            pl.BlockSpec((1, 32, T), lambda b, i: (b, 0, i)),
            pl.BlockSpec((QK, 4), lambda b, i: (0, 0)),
            pl.BlockSpec((1, 32), lambda b, i: (0, 0)),
            pl.BlockSpec((QK, 3), lambda b, i: (0, 0)),
        ],
        out_specs=[
            pl.BlockSpec((1, KNN, T), lambda b, i: (b, 0, i)),
            pl.BlockSpec((1, KNN, T), lambda b, i: (b, 0, i)),
            pl.BlockSpec((1, 1, T), lambda b, i: (b, 0, i)),
            pl.BlockSpec((1, 3 * KNN, T), lambda b, i: (b, 0, i)),
            pl.BlockSpec((QK, 1), lambda b, i: (0, 0)),
            pl.BlockSpec((QK, 1), lambda b, i: (0, 0)),
        ],
        out_shape=(sds((B, KNN, N), jnp.int32), sds((B, KNN, N), f32),
                   sds((B, 1, N), f32), sds((B, 3 * KNN, N), f32),
                   stat, stat),
    )(xyz_hr, xyz_lr, q, k, ybd, foldk, p['bd_w2'], p['rp_w1'])
    sr1, tr1 = _fold(sr, qr, B * N * KNN, p['rp_g1'], p['rp_be1'])

    fold5 = jnp.stack([sr1, tr1, p['rp_b1'], p['rp_b2']], axis=1)
    w2q = p['rp_w2'].T
    q2w = pl.pallas_call(
        _p6_kernel,
        grid=(B, NT),
        in_specs=[
            pl.BlockSpec((1, 3 * KNN, T), lambda b, i: (b, 0, i)),
            pl.BlockSpec((1, KNN, T), lambda b, i: (b, 0, i)),
            pl.BlockSpec((1, KNN, T), lambda b, i: (b, 0, i)),
            pl.BlockSpec((1, QK, T), lambda b, i: (b, 0, i)),
            pl.BlockSpec((1, 6, M), lambda b, i: (b, 0, 0)),
            pl.BlockSpec((QK, 4), lambda b, i: (0, 0)),
            pl.BlockSpec((QK, 3), lambda b, i: (0, 0)),
            pl.BlockSpec((QK, QK), lambda b, i: (0, 0)),
        ],
        out_specs=pl.BlockSpec((1, 6, T), lambda b, i: (b, 0, i)),
        out_shape=sds((B, 6, N), f32),
    )(rel, sg, kidx, q, val_lr, fold5, p['rp_w1'], w2q)

    return (q2w, bdy)
